# Initial kernel scaffold; baseline (speedup 1.0000x reference)
import jax
import jax.numpy as jnp
from jax.experimental import pallas as pl


def kernel(features, eval_locs, eval_indices, quad_weights, W1, W2, W3):
    raise NotImplementedError("write your pallas kernel here")


# speedup vs baseline: 5.1471x; 5.1471x over previous
import functools

import jax
import jax.numpy as jnp
from jax import lax
from jax.experimental import pallas as pl
from jax.experimental.pallas import tpu as pltpu
from jax.experimental.pallas import tpu_sc as plsc

_B = 4
_C = 16
_Q = 10000
_NNZ = 160000
_TN = 1280
_NT = _NNZ // _TN
_VEC = 16
_CH = 16000
_NCH = _NNZ // _CH
_WQL = 5248
_WQS = 4992

_mesh = plsc.VectorSubcoreMesh(core_axis_name="c", subcore_axis_name="s")


@functools.partial(
    pl.kernel,
    out_type=(
        jax.ShapeDtypeStruct((_B * _C * _CH * _NCH,), jnp.float32),
        jax.ShapeDtypeStruct((_NNZ,), jnp.float32),
    ),
    mesh=_mesh,
    scratch_types=[
        pltpu.VMEM((_Q,), jnp.float32),
        pltpu.VMEM((_CH,), jnp.int32),
        pltpu.VMEM((_CH,), jnp.float32),
    ],
    compiler_params=pltpu.CompilerParams(needs_layout_passes=False),
)
def _gather_kernel(feat, qw, idx1, g2_out, wq_out, table, idxbuf, outbuf):
    wid = lax.axis_index("s") * 2 + lax.axis_index("c")

    def _al(off):
        return pl.multiple_of(off, 8)

    @pl.loop(0, 2)
    def _(t):
        p = wid * 2 + t
        b = p // 16
        c = p % 16
        pltpu.sync_copy(feat.at[pl.ds(_al((b * 16 + c) * _Q), _Q)], table)
        base = _al((b * 16 + c) * _NNZ)

        @pl.loop(0, _NCH)
        def _(k):
            pltpu.sync_copy(idx1.at[pl.ds(_al(k * _CH), _CH)], idxbuf)

            @pl.loop(0, _CH // (_VEC * 5))
            def _(v):
                for u in range(5):
                    sl = pl.ds((v * 5 + u) * _VEC, _VEC)
                    outbuf[sl] = plsc.load_gather(table, [idxbuf[sl]])
            pltpu.sync_copy(outbuf, g2_out.at[pl.ds(_al(base + k * _CH), _CH)])

    pltpu.sync_copy(qw, table)
    wstart = _al(wid * _WQS)
    pltpu.sync_copy(idx1.at[pl.ds(wstart, _WQL)], idxbuf.at[pl.ds(0, _WQL)])

    @pl.loop(0, _WQL // (_VEC * 4))
    def _(v):
        for u in range(4):
            sl = pl.ds((v * 4 + u) * _VEC, _VEC)
            outbuf[sl] = plsc.load_gather(table, [idxbuf[sl]])
    pltpu.sync_copy(outbuf.at[pl.ds(0, _WQL)], wq_out.at[pl.ds(wstart, _WQL)])


def _tc_body(locs_ref, g2_ref, w_ref, w1_ref, w2_ref, w3_ref, out_ref):
    lo = locs_ref[...]
    h = jnp.sin(jnp.dot(lo, w1_ref[...], preferred_element_type=jnp.float32))
    h = jnp.sin(jnp.dot(h, w2_ref[...], preferred_element_type=jnp.float32))
    f = jnp.dot(h, w3_ref[...], preferred_element_type=jnp.float32)
    w = w_ref[...]
    for b in range(_B):
        a = g2_ref[b]
        acc = a[:, 0:1] * f[:, 0:16]
        for i in range(1, 16):
            acc = acc + a[:, i:i + 1] * f[:, 16 * i:16 * (i + 1)]
        out_ref[b] = acc * w


def _tc_call(locs, g2, w2d, w1, w2, w3):
    return pl.pallas_call(
        _tc_body,
        grid=(_NT,),
        in_specs=[
            pl.BlockSpec((_TN, 2), lambda t: (t, 0)),
            pl.BlockSpec((_B, _TN, 16), lambda t: (0, t, 0)),
            pl.BlockSpec((_TN, 1), lambda t: (t, 0)),
            pl.BlockSpec((2, 64), lambda t: (0, 0)),
            pl.BlockSpec((64, 64), lambda t: (0, 0)),
            pl.BlockSpec((64, 256), lambda t: (0, 0)),
        ],
        out_specs=pl.BlockSpec((_B, _TN, 16), lambda t: (0, t, 0)),
        out_shape=jax.ShapeDtypeStruct((_B, _NNZ, 16), jnp.float32),
    )(locs, g2, w2d, w1, w2, w3)


@functools.partial(
    pl.kernel,
    out_type=jax.ShapeDtypeStruct((_B * _C * _Q,), jnp.float32),
    mesh=_mesh,
    scratch_types=[
        pltpu.VMEM((_Q,), jnp.float32),
        pltpu.VMEM((_CH,), jnp.int32),
        pltpu.VMEM((_CH,), jnp.float32),
    ],
    compiler_params=pltpu.CompilerParams(needs_layout_passes=False),
)
def _scatter_kernel(vals, idx0, out, acc, idxbuf, valbuf):
    wid = lax.axis_index("s") * 2 + lax.axis_index("c")
    zeros = jnp.zeros((_VEC,), jnp.float32)

    def _al(off):
        return pl.multiple_of(off, 8)

    @pl.loop(0, 2)
    def _(t):
        p = wid * 2 + t
        b = p // 16
        c = p % 16

        @pl.loop(0, _Q // (_VEC * 5))
        def _(v):
            for u in range(5):
                acc[pl.ds((v * 5 + u) * _VEC, _VEC)] = zeros

        base = _al((b * 16 + c) * _NNZ)

        @pl.loop(0, _NCH)
        def _(k):
            pltpu.sync_copy(idx0.at[pl.ds(_al(k * _CH), _CH)], idxbuf)
            pltpu.sync_copy(vals.at[pl.ds(_al(base + k * _CH), _CH)], valbuf)

            @pl.loop(0, _CH // (_VEC * 5))
            def _(v):
                for u in range(5):
                    sl = pl.ds((v * 5 + u) * _VEC, _VEC)
                    plsc.addupdate_scatter(acc, [idxbuf[sl]], valbuf[sl])

        pltpu.sync_copy(acc, out.at[pl.ds(_al((b * 16 + c) * _Q), _Q)])


def kernel(features, eval_locs, eval_indices, quad_weights, W1, W2, W3):
    idx = eval_indices.astype(jnp.int32)
    idx0 = idx[:, 0]
    idx1 = idx[:, 1]

    g2, wq = _gather_kernel(features.reshape(-1), quad_weights, idx1)

    vals = _tc_call(eval_locs, g2.reshape(_B, _NNZ, 16), wq.reshape(_NNZ, 1),
                    W1, W2, W3)

    return _scatter_kernel(vals.reshape(-1), idx0).reshape(_B, _C, _Q)
